# baseline (speedup 1.0000x reference)
import functools

import jax
import jax.numpy as jnp
from jax import lax
from jax.experimental import pallas as pl
from jax.experimental.pallas import tpu as pltpu
from jax.experimental.pallas import tpu_sc as plsc

VOCAB = 1000
N_EMBD = 32
TE = 128
B, T = 1024, 200

NC, NS = 2, 16
NW = NC * NS
BB = 128
NBLK = B // BB
NU = T * NBLK // NW
NR = 2

BB2 = 512


def _emb_gather_body(tab_hbm, idxt_hbm, emb_hbm,
                     idxb0, idxb1, buf0, buf1,
                     semi0, semi1, semg0, semg1, semw0, semw1):
    idxb = (idxb0, idxb1)
    buf = (buf0, buf1)
    semi = (semi0, semi1)
    semg = (semg0, semg1)
    semw = (semw0, semw1)

    wid = lax.axis_index("s") * NC + lax.axis_index("c")
    u0 = wid * NU

    def t_of(u):
        return (u0 + u) // NBLK

    def off_of(u):
        return ((u0 + u) % NBLK) * BB

    def issue_idx(u, k):
        pltpu.async_copy(idxt_hbm.at[t_of(u), pl.ds(off_of(u), BB)], idxb[k],
                         semi[k])

    def wait_idx(u, k):
        pltpu.make_async_copy(idxt_hbm.at[t_of(u), pl.ds(off_of(u), BB)],
                              idxb[k], semi[k]).wait()

    def issue_gather(k):
        pltpu.async_copy(tab_hbm.at[idxb[k]], buf[k], semg[k])

    def wait_gather(k):
        pltpu.make_async_copy(tab_hbm.at[idxb[k]], buf[k], semg[k]).wait()

    def out_ref(u):
        return emb_hbm.at[t_of(u), pl.ds(off_of(u), BB)]

    def wait_write(u, k):
        pltpu.make_async_copy(buf[k], out_ref(u), semw[k]).wait()

    def visit(u, k):
        kn = (k + 1) % NR
        @pl.when(u >= 1)
        def _():
            wait_write(u - 1, kn)

        @pl.when(u + 1 < NU)
        def _():
            wait_idx(u + 1, kn)
            issue_gather(kn)

        wait_gather(k)
        pltpu.async_copy(buf[k], out_ref(u), semw[k])

        @pl.when(u + 2 < NU)
        def _():
            issue_idx(u + 2, k)

    issue_idx(0, 0)
    wait_idx(0, 0)
    issue_gather(0)
    issue_idx(1, 1)

    def group(g, _):
        for k in range(NR):
            visit(NR * g + k, k)
        return 0

    lax.fori_loop(0, NU // NR, group, 0, unroll=False)
    wait_write(NU - 1, (NU - 1) % NR)


@functools.lru_cache(maxsize=1)
def _make_emb_gather():
    mesh = plsc.VectorSubcoreMesh(core_axis_name="c", subcore_axis_name="s")
    return pl.kernel(
        _emb_gather_body,
        mesh=mesh,
        out_type=jax.ShapeDtypeStruct((T, B, TE), jnp.float32),
        scratch_types=(
            [pltpu.VMEM((BB,), jnp.int32) for _ in range(NR)]
            + [pltpu.VMEM((BB, TE), jnp.float32) for _ in range(NR)]
            + [pltpu.SemaphoreType.DMA for _ in range(3 * NR)]
        ),
    )


def _head_body(emb_ref, wt_ref, b_ref, out_ref):
    embc = emb_ref[0][:, :N_EMBD].astype(jnp.bfloat16)
    wt = wt_ref[...].astype(jnp.bfloat16)
    res = lax.dot_general(
        wt, embc, (((1,), (1,)), ((), ())),
        preferred_element_type=jnp.float32,
    )
    out_ref[0] = res + b_ref[...]


def _head(embs, wt, bcol):
    return pl.pallas_call(
        _head_body,
        grid=(T, B // BB2),
        in_specs=[
            pl.BlockSpec((1, BB2, TE), lambda t, i: (t, i, 0)),
            pl.BlockSpec((VOCAB, N_EMBD), lambda t, i: (0, 0)),
            pl.BlockSpec((VOCAB, 1), lambda t, i: (0, 0)),
        ],
        out_specs=pl.BlockSpec((1, VOCAB, BB2), lambda t, i: (t, 0, i)),
        out_shape=jax.ShapeDtypeStruct((T, VOCAB, B), jnp.float32),
    )(embs, wt, bcol)


def kernel(idx, token_embedding_table, lm_head_w, lm_head_b):
    tabp = jnp.pad(token_embedding_table, ((0, 0), (0, TE - N_EMBD)))
    idxt = idx.T
    embs = _make_emb_gather()(tabp, idxt)
    out_t = _head(embs, lm_head_w.T, lm_head_b.reshape(VOCAB, 1))
    return jnp.transpose(out_t, (2, 0, 1))


# speedup vs baseline: 1.2574x; 1.2574x over previous
import functools

import jax
import jax.numpy as jnp
from jax import lax
from jax.experimental import pallas as pl
from jax.experimental.pallas import tpu as pltpu
from jax.experimental.pallas import tpu_sc as plsc

VOCAB = 1000
N_EMBD = 32
TE = 128
B, T = 1024, 200

NC, NS = 2, 16
NW = NC * NS
BB = 128
NBLK = B // BB
NU = T * NBLK // NW
NR = 2

BB2 = 1024


def _emb_gather_body(tab_hbm, idxt_hbm, emb_hbm,
                     idxb0, idxb1, buf0, buf1,
                     semi0, semi1, semg0, semg1, semw0, semw1):
    idxb = (idxb0, idxb1)
    buf = (buf0, buf1)
    semi = (semi0, semi1)
    semg = (semg0, semg1)
    semw = (semw0, semw1)

    wid = lax.axis_index("s") * NC + lax.axis_index("c")
    u0 = wid * NU

    def t_of(u):
        return (u0 + u) // NBLK

    def off_of(u):
        return ((u0 + u) % NBLK) * BB

    def issue_idx(u, k):
        pltpu.async_copy(idxt_hbm.at[t_of(u), pl.ds(off_of(u), BB)], idxb[k],
                         semi[k])

    def wait_idx(u, k):
        pltpu.make_async_copy(idxt_hbm.at[t_of(u), pl.ds(off_of(u), BB)],
                              idxb[k], semi[k]).wait()

    def issue_gather(k):
        pltpu.async_copy(tab_hbm.at[idxb[k]], buf[k], semg[k])

    def wait_gather(k):
        pltpu.make_async_copy(tab_hbm.at[idxb[k]], buf[k], semg[k]).wait()

    def out_ref(u):
        return emb_hbm.at[t_of(u), pl.ds(off_of(u), BB)]

    def wait_write(u, k):
        pltpu.make_async_copy(buf[k], out_ref(u), semw[k]).wait()

    def visit(u, k):
        kn = (k + 1) % NR
        @pl.when(u >= 1)
        def _():
            wait_write(u - 1, kn)

        @pl.when(u + 1 < NU)
        def _():
            wait_idx(u + 1, kn)
            issue_gather(kn)

        wait_gather(k)
        pltpu.async_copy(buf[k], out_ref(u), semw[k])

        @pl.when(u + 2 < NU)
        def _():
            issue_idx(u + 2, k)

    issue_idx(0, 0)
    wait_idx(0, 0)
    issue_gather(0)
    issue_idx(1, 1)

    def group(g, _):
        for k in range(NR):
            visit(NR * g + k, k)
        return 0

    lax.fori_loop(0, NU // NR, group, 0, unroll=False)
    wait_write(NU - 1, (NU - 1) % NR)


@functools.lru_cache(maxsize=1)
def _make_emb_gather():
    mesh = plsc.VectorSubcoreMesh(core_axis_name="c", subcore_axis_name="s")
    return pl.kernel(
        _emb_gather_body,
        mesh=mesh,
        out_type=jax.ShapeDtypeStruct((T, B, TE), jnp.float32),
        scratch_types=(
            [pltpu.VMEM((BB,), jnp.int32) for _ in range(NR)]
            + [pltpu.VMEM((BB, TE), jnp.float32) for _ in range(NR)]
            + [pltpu.SemaphoreType.DMA for _ in range(3 * NR)]
        ),
    )


def _head_body(emb_ref, wt_ref, b_ref, out_ref):
    embc = emb_ref[0][:, :N_EMBD]
    res = lax.dot_general(
        wt_ref[...], embc, (((1,), (1,)), ((), ())),
        preferred_element_type=jnp.float32,
    )
    out_ref[0] = res + b_ref[...]


def _head(embs, wt, bcol):
    return pl.pallas_call(
        _head_body,
        grid=(T, B // BB2),
        in_specs=[
            pl.BlockSpec((1, BB2, TE), lambda t, i: (t, i, 0)),
            pl.BlockSpec((VOCAB, N_EMBD), lambda t, i: (0, 0)),
            pl.BlockSpec((VOCAB, 1), lambda t, i: (0, 0)),
        ],
        out_specs=pl.BlockSpec((1, VOCAB, BB2), lambda t, i: (t, 0, i)),
        out_shape=jax.ShapeDtypeStruct((T, VOCAB, B), jnp.float32),
    )(embs, wt, bcol)


def kernel(idx, token_embedding_table, lm_head_w, lm_head_b):
    tabp = jnp.pad(token_embedding_table, ((0, 0), (0, TE - N_EMBD)))
    idxt = idx.T
    embs = _make_emb_gather()(tabp, idxt)
    out_t = _head(embs, lm_head_w.T, lm_head_b.reshape(VOCAB, 1))
    return jnp.transpose(out_t, (2, 0, 1))
